# baseline (speedup 1.0000x reference)
import numpy as np
import jax
import jax.numpy as jnp
from jax.experimental import pallas as pl
from jax.experimental.pallas import tpu as pltpu

_FEATURE_STRIDES = [4, 8, 16, 32, 64]
_ANCHOR_SIZES = [32, 64, 128, 256, 512]
_ANCHOR_RATIOS = [0.5, 1.0, 2.0]
_FEATURE_SHAPES_STATIC = [[128, 128], [64, 64], [32, 32], [16, 16], [8, 8]]

_A_REAL = 65472
_ROWS = 512
_LANES = 128
_A_PAD = _ROWS * _LANES
_G = 20
_NUM_FG = 128
_NUM_SAMPLES = 256
_OV_POS = 0.7
_OV_NEG = 0.3


def _anchor_planes_np():
    all_a = []
    for (H, W), stride, size in zip(_FEATURE_SHAPES_STATIC, _FEATURE_STRIDES,
                                    _ANCHOR_SIZES):
        base = []
        for r in _ANCHOR_RATIOS:
            w = size / np.sqrt(r)
            h = size * np.sqrt(r)
            base.append([-w / 2.0, -h / 2.0, w / 2.0, h / 2.0])
        base = np.asarray(base, dtype=np.float32)
        sx = (np.arange(int(W)) + 0.5) * stride
        sy = (np.arange(int(H)) + 0.5) * stride
        cx, cy = np.meshgrid(sx, sy)
        shifts = np.stack([cx.ravel(), cy.ravel(), cx.ravel(), cy.ravel()],
                          axis=1).astype(np.float32)
        a = (shifts[:, None, :] + base[None, :, :]).reshape(-1, 4)
        all_a.append(a)
    anchors = np.concatenate(all_a, axis=0).astype(np.float32)
    pad = np.tile(np.array([[0.0, 0.0, 1.0, 1.0]], np.float32),
                  (_A_PAD - _A_REAL, 1))
    anchors = np.concatenate([anchors, pad], axis=0)
    return np.ascontiguousarray(
        anchors.reshape(_ROWS, _LANES, 4).transpose(2, 0, 1))


_ANCHOR_PLANES = _anchor_planes_np()


def _body(anch_ref, gt_ref, labels_ref, reg_ref, score_ref,
          dx_ref, dy_ref, dw_ref, dh_ref):
    f32 = jnp.float32
    i32 = jnp.int32
    ax1 = anch_ref[0]
    ay1 = anch_ref[1]
    ax2 = anch_ref[2]
    ay2 = anch_ref[3]
    area_a = (ax2 - ax1) * (ay2 - ay1)

    row_i = jax.lax.broadcasted_iota(i32, (_ROWS, _LANES), 0)
    col_i = jax.lax.broadcasted_iota(i32, (_ROWS, _LANES), 1)
    lin = row_i * _LANES + col_i
    valid = lin < _A_REAL

    max_ov = jnp.full((_ROWS, _LANES), -1.0, f32)
    mgx1 = jnp.zeros((_ROWS, _LANES), f32)
    mgy1 = jnp.zeros((_ROWS, _LANES), f32)
    mgx2 = jnp.zeros((_ROWS, _LANES), f32)
    mgy2 = jnp.zeros((_ROWS, _LANES), f32)
    best_idx = []
    for g in range(_G):
        gx1 = gt_ref[0, 0, 4 * g + 0]
        gy1 = gt_ref[0, 0, 4 * g + 1]
        gx2 = gt_ref[0, 0, 4 * g + 2]
        gy2 = gt_ref[0, 0, 4 * g + 3]
        area_b = (gx2 - gx1) * (gy2 - gy1)
        w = jnp.clip(jnp.minimum(ax2, gx2) - jnp.maximum(ax1, gx1), 0.0)
        h = jnp.clip(jnp.minimum(ay2, gy2) - jnp.maximum(ay1, gy1), 0.0)
        inter = w * h
        union = (area_a + area_b) - inter
        iou = inter / jnp.maximum(union, 1e-9)
        iou = jnp.where(valid, iou, -1.0)
        m_g = jnp.max(iou)
        b_g = jnp.min(jnp.where(iou == m_g, lin, _A_PAD))
        best_idx.append(b_g)
        upd = iou > max_ov
        max_ov = jnp.where(upd, iou, max_ov)
        mgx1 = jnp.where(upd, gx1, mgx1)
        mgy1 = jnp.where(upd, gy1, mgy1)
        mgx2 = jnp.where(upd, gx2, mgx2)
        mgy2 = jnp.where(upd, gy2, mgy2)

    forced = lin == best_idx[0]
    for g in range(1, _G):
        forced = forced | (lin == best_idx[g])

    pos_score = jnp.where(forced, 2.0,
                          jnp.where(max_ov > _OV_POS, max_ov, -1.0))
    num_pos_all = jnp.sum((pos_score > 0.0).astype(i32))
    n_pos = jnp.minimum(num_pos_all, _NUM_FG)
    num_bg = _NUM_SAMPLES - n_pos

    neg = valid & (max_ov < _OV_NEG) & (pos_score <= 0.0)
    total_neg = jnp.sum(neg.astype(i32))
    nb = jnp.minimum(num_bg, total_neg)

    key = 1.0 - max_ov
    kbits = jax.lax.bitcast_convert_type(key, i32)
    mbits = jnp.where(neg, kbits, -1)

    def bs_body(_, c):
        lo, hi = c
        mid = lo + ((hi - lo + 1) // 2)
        cnt = jnp.sum((mbits >= mid).astype(i32))
        take = cnt >= nb
        return (jnp.where(take, mid, lo), jnp.where(take, hi, mid - 1))

    t, _ = jax.lax.fori_loop(0, 31, bs_body,
                             (jnp.asarray(0, i32), jnp.asarray(1 << 30, i32)))

    hi_mask = mbits > t
    c_more = jnp.sum(hi_mask.astype(i32))
    need = (nb - c_more).astype(f32)
    ties = (mbits == t).astype(f32)
    up_incl = (jax.lax.broadcasted_iota(i32, (_LANES, _LANES), 0) <=
               jax.lax.broadcasted_iota(i32, (_LANES, _LANES), 1)).astype(f32)
    within_incl = jax.lax.dot(ties, up_incl,
                              preferred_element_type=f32)
    row_tot = within_incl[:, _LANES - 1:_LANES]
    strict_lo = (jax.lax.broadcasted_iota(i32, (_ROWS, _ROWS), 1) <
                 jax.lax.broadcasted_iota(i32, (_ROWS, _ROWS), 0)).astype(f32)
    row_prefix = jax.lax.dot(strict_lo, row_tot,
                             preferred_element_type=f32)
    rank = row_prefix + (within_incl - ties)
    label0 = hi_mask | ((ties > 0.0) & (rank < need))

    aw = ax2 - ax1
    ah = ay2 - ay1
    axc = (ax1 + ax2) * 0.5
    ayc = (ay1 + ay2) * 0.5
    gw = mgx2 - mgx1
    gh = mgy2 - mgy1
    gxc = (mgx1 + mgx2) * 0.5
    gyc = (mgy1 + mgy2) * 0.5
    dx = (gxc - axc) / aw
    dy = (gyc - ayc) / ah
    dw = jnp.log(jnp.maximum(gw, 1e-6) / aw)
    dh = jnp.log(jnp.maximum(gh, 1e-6) / ah)

    score_ref[:, :] = pos_score
    dx_ref[:, :] = dx
    dy_ref[:, :] = dy
    dw_ref[:, :] = dw
    dh_ref[:, :] = dh
    reg_ref[0] = jnp.zeros((_NUM_FG, 4), f32)

    slot_i = jax.lax.broadcasted_iota(i32, (_NUM_FG, 1), 0)
    comp_i = jax.lax.broadcasted_iota(i32, (1, 4), 1)
    iota_r = jax.lax.broadcasted_iota(i32, (1, _ROWS), 1)
    iota_c = jax.lax.broadcasted_iota(i32, (1, _LANES), 1)

    rm0 = jnp.transpose(jnp.max(pos_score, axis=1, keepdims=True))
    m0 = jnp.max(rm0)
    r0 = jnp.min(jnp.where(rm0 == m0, iota_r, _ROWS))

    def cond(c):
        p, m, _, _ = c
        return (p < _NUM_FG) & (m > 0.0)

    def body(c):
        p, m, r, rm = c
        row = score_ref[pl.ds(r, 1), :]
        ccol = jnp.min(jnp.where(row == m, iota_c, _LANES))
        selc = iota_c == ccol
        v_dx = jnp.sum(jnp.where(selc, dx_ref[pl.ds(r, 1), :], 0.0))
        v_dy = jnp.sum(jnp.where(selc, dy_ref[pl.ds(r, 1), :], 0.0))
        v_dw = jnp.sum(jnp.where(selc, dw_ref[pl.ds(r, 1), :], 0.0))
        v_dh = jnp.sum(jnp.where(selc, dh_ref[pl.ds(r, 1), :], 0.0))
        rowv = (v_dx * (comp_i == 0) + v_dy * (comp_i == 1) +
                v_dw * (comp_i == 2) + v_dh * (comp_i == 3)).astype(f32)
        oh = (slot_i == p).astype(f32)
        reg_ref[0] = reg_ref[0] + oh * rowv
        newrow = jnp.where(selc, -2.0, row)
        score_ref[pl.ds(r, 1), :] = newrow
        rm2 = jnp.where(iota_r == r, jnp.max(newrow), rm)
        m2 = jnp.max(rm2)
        r2 = jnp.min(jnp.where(rm2 == m2, iota_r, _ROWS))
        return (p + 1, m2, r2, rm2)

    jax.lax.while_loop(cond, body, (jnp.asarray(0, i32), m0, r0, rm0))

    sel = score_ref[:, :] == -2.0
    labels_ref[0] = jnp.where(sel, 1,
                              jnp.where(label0, 0, -1)).astype(i32)


def kernel(gt_bboxes, feature_shapes):
    del feature_shapes
    B = gt_bboxes.shape[0]
    gt_flat = gt_bboxes.reshape(B, 1, 4 * _G).astype(jnp.float32)
    planes = jnp.asarray(_ANCHOR_PLANES)

    labels3, reg = pl.pallas_call(
        _body,
        grid=(B,),
        in_specs=[
            pl.BlockSpec((4, _ROWS, _LANES), lambda b: (0, 0, 0)),
            pl.BlockSpec((1, 1, 4 * _G), lambda b: (b, 0, 0),
                         memory_space=pltpu.SMEM),
        ],
        out_specs=[
            pl.BlockSpec((1, _ROWS, _LANES), lambda b: (b, 0, 0)),
            pl.BlockSpec((1, _NUM_FG, 4), lambda b: (b, 0, 0)),
        ],
        out_shape=[
            jax.ShapeDtypeStruct((B, _ROWS, _LANES), jnp.int32),
            jax.ShapeDtypeStruct((B, _NUM_FG, 4), jnp.float32),
        ],
        scratch_shapes=[pltpu.VMEM((_ROWS, _LANES), jnp.float32)
                        for _ in range(5)],
        compiler_params=pltpu.CompilerParams(
            dimension_semantics=("arbitrary",)),
    )(planes, gt_flat)

    labels = labels3.reshape(B, _A_PAD)[:, :_A_REAL].astype(jnp.int8)
    return labels, reg


# speedup vs baseline: 1.1892x; 1.1892x over previous
import numpy as np
import jax
import jax.numpy as jnp
from jax.experimental import pallas as pl
from jax.experimental.pallas import tpu as pltpu

_FEATURE_STRIDES = [4, 8, 16, 32, 64]
_ANCHOR_SIZES = [32, 64, 128, 256, 512]
_ANCHOR_RATIOS = [0.5, 1.0, 2.0]
_FEATURE_SHAPES_STATIC = [[128, 128], [64, 64], [32, 32], [16, 16], [8, 8]]

_A_REAL = 65472
_ROWS = 512
_LANES = 128
_A_PAD = _ROWS * _LANES
_B = 8
_G = 20
_NUM_FG = 128
_NUM_SAMPLES = 256
_OV_POS = 0.7
_OV_NEG = 0.3


def _anchor_planes_np():
    all_a = []
    for (H, W), stride, size in zip(_FEATURE_SHAPES_STATIC, _FEATURE_STRIDES,
                                    _ANCHOR_SIZES):
        base = []
        for r in _ANCHOR_RATIOS:
            w = size / np.sqrt(r)
            h = size * np.sqrt(r)
            base.append([-w / 2.0, -h / 2.0, w / 2.0, h / 2.0])
        base = np.asarray(base, dtype=np.float32)
        sx = (np.arange(int(W)) + 0.5) * stride
        sy = (np.arange(int(H)) + 0.5) * stride
        cx, cy = np.meshgrid(sx, sy)
        shifts = np.stack([cx.ravel(), cy.ravel(), cx.ravel(), cy.ravel()],
                          axis=1).astype(np.float32)
        a = (shifts[:, None, :] + base[None, :, :]).reshape(-1, 4)
        all_a.append(a)
    anchors = np.concatenate(all_a, axis=0).astype(np.float32)
    pad = np.tile(np.array([[0.0, 0.0, 1.0, 1.0]], np.float32),
                  (_A_PAD - _A_REAL, 1))
    anchors = np.concatenate([anchors, pad], axis=0)
    return np.ascontiguousarray(
        anchors.reshape(_ROWS, _LANES, 4).transpose(2, 0, 1))


_ANCHOR_PLANES = _anchor_planes_np()


def _body(anch_ref, gt_ref, labels_ref, reg_ref, score_ref,
          dx_ref, dy_ref, dw_ref, dh_ref):
    f32 = jnp.float32
    i32 = jnp.int32
    ax1 = anch_ref[0]
    ay1 = anch_ref[1]
    ax2 = anch_ref[2]
    ay2 = anch_ref[3]
    area_a = (ax2 - ax1) * (ay2 - ay1)
    aw = ax2 - ax1
    ah = ay2 - ay1
    axc = (ax1 + ax2) * 0.5
    ayc = (ay1 + ay2) * 0.5

    row_i = jax.lax.broadcasted_iota(i32, (_ROWS, _LANES), 0)
    col_i = jax.lax.broadcasted_iota(i32, (_ROWS, _LANES), 1)
    lin = row_i * _LANES + col_i
    valid = lin < _A_REAL

    up_incl = (jax.lax.broadcasted_iota(i32, (_LANES, _LANES), 0) <=
               jax.lax.broadcasted_iota(i32, (_LANES, _LANES), 1)).astype(f32)
    strict_lo = (jax.lax.broadcasted_iota(i32, (_ROWS, _ROWS), 1) <
                 jax.lax.broadcasted_iota(i32, (_ROWS, _ROWS), 0)).astype(f32)

    mbits_l = []
    nb_l = []
    rm_l = []
    for b in range(_B):
        max_ov = jnp.full((_ROWS, _LANES), -1.0, f32)
        mgx1 = jnp.zeros((_ROWS, _LANES), f32)
        mgy1 = jnp.zeros((_ROWS, _LANES), f32)
        mgx2 = jnp.zeros((_ROWS, _LANES), f32)
        mgy2 = jnp.zeros((_ROWS, _LANES), f32)
        best_idx = []
        for g in range(_G):
            gx1 = gt_ref[b, 0, 4 * g + 0]
            gy1 = gt_ref[b, 0, 4 * g + 1]
            gx2 = gt_ref[b, 0, 4 * g + 2]
            gy2 = gt_ref[b, 0, 4 * g + 3]
            area_b = (gx2 - gx1) * (gy2 - gy1)
            w = jnp.clip(jnp.minimum(ax2, gx2) - jnp.maximum(ax1, gx1), 0.0)
            h = jnp.clip(jnp.minimum(ay2, gy2) - jnp.maximum(ay1, gy1), 0.0)
            inter = w * h
            union = (area_a + area_b) - inter
            iou = inter / jnp.maximum(union, 1e-9)
            iou = jnp.where(valid, iou, -1.0)
            m_g = jnp.max(iou)
            b_g = jnp.min(jnp.where(iou == m_g, lin, _A_PAD))
            best_idx.append(b_g)
            upd = iou > max_ov
            max_ov = jnp.where(upd, iou, max_ov)
            mgx1 = jnp.where(upd, gx1, mgx1)
            mgy1 = jnp.where(upd, gy1, mgy1)
            mgx2 = jnp.where(upd, gx2, mgx2)
            mgy2 = jnp.where(upd, gy2, mgy2)

        forced = lin == best_idx[0]
        for g in range(1, _G):
            forced = forced | (lin == best_idx[g])

        pos_score = jnp.where(forced, 2.0,
                              jnp.where(max_ov > _OV_POS, max_ov, -1.0))
        num_pos_all = jnp.sum((pos_score > 0.0).astype(i32))
        n_pos = jnp.minimum(num_pos_all, _NUM_FG)
        num_bg = _NUM_SAMPLES - n_pos

        neg = valid & (max_ov < _OV_NEG) & (pos_score <= 0.0)
        total_neg = jnp.sum(neg.astype(i32))
        nb_l.append(jnp.minimum(num_bg, total_neg))

        key = 1.0 - max_ov
        kbits = jax.lax.bitcast_convert_type(key, i32)
        mbits_l.append(jnp.where(neg, kbits, -1))

        gw = mgx2 - mgx1
        gh = mgy2 - mgy1
        gxc = (mgx1 + mgx2) * 0.5
        gyc = (mgy1 + mgy2) * 0.5
        score_ref[pl.ds(b * _ROWS, _ROWS), :] = pos_score
        dx_ref[pl.ds(b * _ROWS, _ROWS), :] = (gxc - axc) / aw
        dy_ref[pl.ds(b * _ROWS, _ROWS), :] = (gyc - ayc) / ah
        dw_ref[pl.ds(b * _ROWS, _ROWS), :] = jnp.log(
            jnp.maximum(gw, 1e-6) / aw)
        dh_ref[pl.ds(b * _ROWS, _ROWS), :] = jnp.log(
            jnp.maximum(gh, 1e-6) / ah)
        rm_l.append(jnp.transpose(
            jnp.max(pos_score, axis=1, keepdims=True)))

    def bs_body(_, c):
        los, his = c
        nlos = []
        nhis = []
        for b in range(_B):
            mid = los[b] + ((his[b] - los[b] + 1) // 2)
            cnt = jnp.sum((mbits_l[b] >= mid).astype(i32))
            take = cnt >= nb_l[b]
            nlos.append(jnp.where(take, mid, los[b]))
            nhis.append(jnp.where(take, his[b], mid - 1))
        return (tuple(nlos), tuple(nhis))

    zeros8 = tuple(jnp.asarray(0, i32) for _ in range(_B))
    tops8 = tuple(jnp.asarray(1 << 30, i32) for _ in range(_B))
    ts, _ = jax.lax.fori_loop(0, 31, bs_body, (zeros8, tops8))

    label0_l = []
    for b in range(_B):
        t = ts[b]
        mbits = mbits_l[b]
        hi_mask = mbits > t
        c_more = jnp.sum(hi_mask.astype(i32))
        need = (nb_l[b] - c_more).astype(f32)
        ties = (mbits == t).astype(f32)
        within_incl = jax.lax.dot(ties, up_incl, preferred_element_type=f32)
        row_tot = within_incl[:, _LANES - 1:_LANES]
        row_prefix = jax.lax.dot(strict_lo, row_tot,
                                 preferred_element_type=f32)
        rank = row_prefix + (within_incl - ties)
        label0_l.append(hi_mask | ((ties > 0.0) & (rank < need)))

    for b in range(_B):
        reg_ref[b] = jnp.zeros((_NUM_FG, 4), f32)

    slot_i = jax.lax.broadcasted_iota(i32, (_NUM_FG, 1), 0)
    comp_i = jax.lax.broadcasted_iota(i32, (1, 4), 1)
    iota_r = jax.lax.broadcasted_iota(i32, (1, _ROWS), 1)
    iota_c = jax.lax.broadcasted_iota(i32, (1, _LANES), 1)

    ms0 = []
    rs0 = []
    for b in range(_B):
        m0 = jnp.max(rm_l[b])
        ms0.append(m0)
        rs0.append(jnp.min(jnp.where(rm_l[b] == m0, iota_r, _ROWS - 1)))

    def cond(c):
        ps, ms, _, _ = c
        alive = (ps[0] < _NUM_FG) & (ms[0] > 0.0)
        for b in range(1, _B):
            alive = alive | ((ps[b] < _NUM_FG) & (ms[b] > 0.0))
        return alive

    def body(c):
        ps, ms, rs, rms = c
        nps = []
        nms = []
        nrs = []
        nrms = []
        for b in range(_B):
            p, m, r, rm = ps[b], ms[b], rs[b], rms[b]
            active = (p < _NUM_FG) & (m > 0.0)
            base = b * _ROWS
            row = score_ref[pl.ds(base + r, 1), :]
            ccol = jnp.min(jnp.where(row == m, iota_c, _LANES - 1))
            selc = iota_c == ccol
            v_dx = jnp.sum(jnp.where(selc, dx_ref[pl.ds(base + r, 1), :], 0.0))
            v_dy = jnp.sum(jnp.where(selc, dy_ref[pl.ds(base + r, 1), :], 0.0))
            v_dw = jnp.sum(jnp.where(selc, dw_ref[pl.ds(base + r, 1), :], 0.0))
            v_dh = jnp.sum(jnp.where(selc, dh_ref[pl.ds(base + r, 1), :], 0.0))
            rowv = (v_dx * (comp_i == 0) + v_dy * (comp_i == 1) +
                    v_dw * (comp_i == 2) + v_dh * (comp_i == 3)).astype(f32)
            oh = (slot_i == p).astype(f32)
            act_f = jnp.where(active, 1.0, 0.0)
            reg_ref[b] = reg_ref[b] + act_f * (oh * rowv)
            newrow = jnp.where(selc & active, -2.0, row)
            score_ref[pl.ds(base + r, 1), :] = newrow
            rm2 = jnp.where((iota_r == r) & active, jnp.max(newrow), rm)
            m2 = jnp.max(rm2)
            r2 = jnp.min(jnp.where(rm2 == m2, iota_r, _ROWS - 1))
            nps.append(p + jnp.where(active, 1, 0))
            nms.append(m2)
            nrs.append(r2)
            nrms.append(rm2)
        return (tuple(nps), tuple(nms), tuple(nrs), tuple(nrms))

    zero_ps = tuple(jnp.asarray(0, i32) for _ in range(_B))
    jax.lax.while_loop(cond, body,
                       (zero_ps, tuple(ms0), tuple(rs0), tuple(rm_l)))

    for b in range(_B):
        sel = score_ref[pl.ds(b * _ROWS, _ROWS), :] == -2.0
        labels_ref[b] = jnp.where(sel, 1,
                                  jnp.where(label0_l[b], 0, -1)).astype(i32)


def kernel(gt_bboxes, feature_shapes):
    del feature_shapes
    B = gt_bboxes.shape[0]
    gt_flat = gt_bboxes.reshape(B, 1, 4 * _G).astype(jnp.float32)
    planes = jnp.asarray(_ANCHOR_PLANES)

    labels3, reg = pl.pallas_call(
        _body,
        grid=(1,),
        in_specs=[
            pl.BlockSpec((4, _ROWS, _LANES), lambda i: (0, 0, 0)),
            pl.BlockSpec((B, 1, 4 * _G), lambda i: (0, 0, 0),
                         memory_space=pltpu.SMEM),
        ],
        out_specs=[
            pl.BlockSpec((B, _ROWS, _LANES), lambda i: (0, 0, 0)),
            pl.BlockSpec((B, _NUM_FG, 4), lambda i: (0, 0, 0)),
        ],
        out_shape=[
            jax.ShapeDtypeStruct((B, _ROWS, _LANES), jnp.int32),
            jax.ShapeDtypeStruct((B, _NUM_FG, 4), jnp.float32),
        ],
        scratch_shapes=[pltpu.VMEM((_B * _ROWS, _LANES), jnp.float32)
                        for _ in range(5)],
        compiler_params=pltpu.CompilerParams(
            dimension_semantics=("arbitrary",)),
    )(planes, gt_flat)

    labels = labels3.reshape(B, _A_PAD)[:, :_A_REAL].astype(jnp.int8)
    return labels, reg


# speedup vs baseline: 1.2014x; 1.0103x over previous
import numpy as np
import jax
import jax.numpy as jnp
from jax.experimental import pallas as pl
from jax.experimental.pallas import tpu as pltpu

_FEATURE_STRIDES = [4, 8, 16, 32, 64]
_ANCHOR_SIZES = [32, 64, 128, 256, 512]
_ANCHOR_RATIOS = [0.5, 1.0, 2.0]
_FEATURE_SHAPES_STATIC = [[128, 128], [64, 64], [32, 32], [16, 16], [8, 8]]

_A_REAL = 65472
_ROWS = 512
_LANES = 128
_A_PAD = _ROWS * _LANES
_B = 8
_G = 20
_NUM_FG = 128
_NUM_SAMPLES = 256
_OV_POS = 0.7
_OV_NEG = 0.3


def _anchor_planes_np():
    all_a = []
    for (H, W), stride, size in zip(_FEATURE_SHAPES_STATIC, _FEATURE_STRIDES,
                                    _ANCHOR_SIZES):
        base = []
        for r in _ANCHOR_RATIOS:
            w = size / np.sqrt(r)
            h = size * np.sqrt(r)
            base.append([-w / 2.0, -h / 2.0, w / 2.0, h / 2.0])
        base = np.asarray(base, dtype=np.float32)
        sx = (np.arange(int(W)) + 0.5) * stride
        sy = (np.arange(int(H)) + 0.5) * stride
        cx, cy = np.meshgrid(sx, sy)
        shifts = np.stack([cx.ravel(), cy.ravel(), cx.ravel(), cy.ravel()],
                          axis=1).astype(np.float32)
        a = (shifts[:, None, :] + base[None, :, :]).reshape(-1, 4)
        all_a.append(a)
    anchors = np.concatenate(all_a, axis=0).astype(np.float32)
    pad = np.tile(np.array([[0.0, 0.0, 1.0, 1.0]], np.float32),
                  (_A_PAD - _A_REAL, 1))
    anchors = np.concatenate([anchors, pad], axis=0)
    return np.ascontiguousarray(
        anchors.reshape(_ROWS, _LANES, 4).transpose(2, 0, 1))


_ANCHOR_PLANES = _anchor_planes_np()


def _body(anch_ref, gt_ref, labels_ref, reg_ref, score_ref,
          dx_ref, dy_ref, dw_ref, dh_ref):
    f32 = jnp.float32
    i32 = jnp.int32
    ax1 = anch_ref[0]
    ay1 = anch_ref[1]
    ax2 = anch_ref[2]
    ay2 = anch_ref[3]
    area_a = (ax2 - ax1) * (ay2 - ay1)
    aw = ax2 - ax1
    ah = ay2 - ay1
    axc = (ax1 + ax2) * 0.5
    ayc = (ay1 + ay2) * 0.5

    row_i = jax.lax.broadcasted_iota(i32, (_ROWS, _LANES), 0)
    col_i = jax.lax.broadcasted_iota(i32, (_ROWS, _LANES), 1)
    lin = row_i * _LANES + col_i
    valid = lin < _A_REAL

    up_incl = (jax.lax.broadcasted_iota(i32, (_LANES, _LANES), 0) <=
               jax.lax.broadcasted_iota(i32, (_LANES, _LANES), 1)).astype(f32)
    strict_lo = (jax.lax.broadcasted_iota(i32, (_ROWS, _ROWS), 1) <
                 jax.lax.broadcasted_iota(i32, (_ROWS, _ROWS), 0)).astype(f32)

    mbits_l = []
    nb_l = []
    rm_l = []
    for b in range(_B):
        max_ov = jnp.full((_ROWS, _LANES), -1.0, f32)
        mgx1 = jnp.zeros((_ROWS, _LANES), f32)
        mgy1 = jnp.zeros((_ROWS, _LANES), f32)
        mgx2 = jnp.zeros((_ROWS, _LANES), f32)
        mgy2 = jnp.zeros((_ROWS, _LANES), f32)
        best_idx = []
        for g in range(_G):
            gx1 = gt_ref[b, 0, 4 * g + 0]
            gy1 = gt_ref[b, 0, 4 * g + 1]
            gx2 = gt_ref[b, 0, 4 * g + 2]
            gy2 = gt_ref[b, 0, 4 * g + 3]
            area_b = (gx2 - gx1) * (gy2 - gy1)
            w = jnp.clip(jnp.minimum(ax2, gx2) - jnp.maximum(ax1, gx1), 0.0)
            h = jnp.clip(jnp.minimum(ay2, gy2) - jnp.maximum(ay1, gy1), 0.0)
            inter = w * h
            union = (area_a + area_b) - inter
            iou = inter / jnp.maximum(union, 1e-9)
            iou = jnp.where(valid, iou, -1.0)
            m_g = jnp.max(iou)
            b_g = jnp.min(jnp.where(iou == m_g, lin, _A_PAD))
            best_idx.append(b_g)
            upd = iou > max_ov
            max_ov = jnp.where(upd, iou, max_ov)
            mgx1 = jnp.where(upd, gx1, mgx1)
            mgy1 = jnp.where(upd, gy1, mgy1)
            mgx2 = jnp.where(upd, gx2, mgx2)
            mgy2 = jnp.where(upd, gy2, mgy2)

        forced = lin == best_idx[0]
        for g in range(1, _G):
            forced = forced | (lin == best_idx[g])

        pos_score = jnp.where(forced, 2.0,
                              jnp.where(max_ov > _OV_POS, max_ov, -1.0))
        num_pos_all = jnp.sum((pos_score > 0.0).astype(i32))
        n_pos = jnp.minimum(num_pos_all, _NUM_FG)
        num_bg = _NUM_SAMPLES - n_pos

        neg = valid & (max_ov < _OV_NEG) & (pos_score <= 0.0)
        total_neg = jnp.sum(neg.astype(i32))
        nb_l.append(jnp.minimum(num_bg, total_neg))

        key = 1.0 - max_ov
        kbits = jax.lax.bitcast_convert_type(key, i32)
        mbits_l.append(jnp.where(neg, kbits, -1))

        gw = mgx2 - mgx1
        gh = mgy2 - mgy1
        gxc = (mgx1 + mgx2) * 0.5
        gyc = (mgy1 + mgy2) * 0.5
        score_ref[pl.ds(b * _ROWS, _ROWS), :] = pos_score
        dx_ref[pl.ds(b * _ROWS, _ROWS), :] = (gxc - axc) / aw
        dy_ref[pl.ds(b * _ROWS, _ROWS), :] = (gyc - ayc) / ah
        dw_ref[pl.ds(b * _ROWS, _ROWS), :] = jnp.log(
            jnp.maximum(gw, 1e-6) / aw)
        dh_ref[pl.ds(b * _ROWS, _ROWS), :] = jnp.log(
            jnp.maximum(gh, 1e-6) / ah)
        rm_l.append(jnp.transpose(
            jnp.max(pos_score, axis=1, keepdims=True)))

    def bs_body(_, c):
        los, his = c
        nlos = []
        nhis = []
        for b in range(_B):
            mid = los[b] + ((his[b] - los[b] + 1) // 2)
            cnt = jnp.sum((mbits_l[b] >= mid).astype(i32))
            take = cnt >= nb_l[b]
            nlos.append(jnp.where(take, mid, los[b]))
            nhis.append(jnp.where(take, his[b], mid - 1))
        return (tuple(nlos), tuple(nhis))

    zeros8 = tuple(jnp.asarray(0x3F333333, i32) for _ in range(_B))
    tops8 = tuple(jnp.asarray(0x3F800000, i32) for _ in range(_B))
    ts, _ = jax.lax.fori_loop(0, 23, bs_body, (zeros8, tops8))

    label0_l = []
    for b in range(_B):
        t = ts[b]
        mbits = mbits_l[b]
        hi_mask = mbits > t
        c_more = jnp.sum(hi_mask.astype(i32))
        need = (nb_l[b] - c_more).astype(f32)
        ties = (mbits == t).astype(f32)
        within_incl = jax.lax.dot(ties, up_incl, preferred_element_type=f32)
        row_tot = within_incl[:, _LANES - 1:_LANES]
        row_prefix = jax.lax.dot(strict_lo, row_tot,
                                 preferred_element_type=f32)
        rank = row_prefix + (within_incl - ties)
        label0_l.append(hi_mask | ((ties > 0.0) & (rank < need)))

    for b in range(_B):
        reg_ref[b] = jnp.zeros((_NUM_FG, 4), f32)

    slot_i = jax.lax.broadcasted_iota(i32, (_NUM_FG, 1), 0)
    comp_i = jax.lax.broadcasted_iota(i32, (1, 4), 1)
    iota_r = jax.lax.broadcasted_iota(i32, (1, _ROWS), 1)
    iota_c = jax.lax.broadcasted_iota(i32, (1, _LANES), 1)

    ms0 = []
    rs0 = []
    for b in range(_B):
        m0 = jnp.max(rm_l[b])
        ms0.append(m0)
        rs0.append(jnp.min(jnp.where(rm_l[b] == m0, iota_r, _ROWS - 1)))

    def cond(c):
        ps, ms, _, _ = c
        alive = (ps[0] < _NUM_FG) & (ms[0] > 0.0)
        for b in range(1, _B):
            alive = alive | ((ps[b] < _NUM_FG) & (ms[b] > 0.0))
        return alive

    def body(c):
        ps, ms, rs, rms = c
        nps = []
        nms = []
        nrs = []
        nrms = []
        for b in range(_B):
            p, m, r, rm = ps[b], ms[b], rs[b], rms[b]
            active = (p < _NUM_FG) & (m > 0.0)
            base = b * _ROWS
            row = score_ref[pl.ds(base + r, 1), :]
            ccol = jnp.min(jnp.where(row == m, iota_c, _LANES - 1))
            selc = iota_c == ccol
            v_dx = jnp.sum(jnp.where(selc, dx_ref[pl.ds(base + r, 1), :], 0.0))
            v_dy = jnp.sum(jnp.where(selc, dy_ref[pl.ds(base + r, 1), :], 0.0))
            v_dw = jnp.sum(jnp.where(selc, dw_ref[pl.ds(base + r, 1), :], 0.0))
            v_dh = jnp.sum(jnp.where(selc, dh_ref[pl.ds(base + r, 1), :], 0.0))
            rowv = (v_dx * (comp_i == 0) + v_dy * (comp_i == 1) +
                    v_dw * (comp_i == 2) + v_dh * (comp_i == 3)).astype(f32)
            oh = (slot_i == p).astype(f32)
            act_f = jnp.where(active, 1.0, 0.0)
            reg_ref[b] = reg_ref[b] + act_f * (oh * rowv)
            newrow = jnp.where(selc & active, -2.0, row)
            score_ref[pl.ds(base + r, 1), :] = newrow
            rm2 = jnp.where((iota_r == r) & active, jnp.max(newrow), rm)
            m2 = jnp.max(rm2)
            r2 = jnp.min(jnp.where(rm2 == m2, iota_r, _ROWS - 1))
            nps.append(p + jnp.where(active, 1, 0))
            nms.append(m2)
            nrs.append(r2)
            nrms.append(rm2)
        return (tuple(nps), tuple(nms), tuple(nrs), tuple(nrms))

    zero_ps = tuple(jnp.asarray(0, i32) for _ in range(_B))
    jax.lax.while_loop(cond, body,
                       (zero_ps, tuple(ms0), tuple(rs0), tuple(rm_l)))

    for b in range(_B):
        sel = score_ref[pl.ds(b * _ROWS, _ROWS), :] == -2.0
        labels_ref[b] = jnp.where(sel, 1,
                                  jnp.where(label0_l[b], 0, -1)).astype(i32)


def kernel(gt_bboxes, feature_shapes):
    del feature_shapes
    B = gt_bboxes.shape[0]
    gt_flat = gt_bboxes.reshape(B, 1, 4 * _G).astype(jnp.float32)
    planes = jnp.asarray(_ANCHOR_PLANES)

    labels3, reg = pl.pallas_call(
        _body,
        grid=(1,),
        in_specs=[
            pl.BlockSpec((4, _ROWS, _LANES), lambda i: (0, 0, 0)),
            pl.BlockSpec((B, 1, 4 * _G), lambda i: (0, 0, 0),
                         memory_space=pltpu.SMEM),
        ],
        out_specs=[
            pl.BlockSpec((B, _ROWS, _LANES), lambda i: (0, 0, 0)),
            pl.BlockSpec((B, _NUM_FG, 4), lambda i: (0, 0, 0)),
        ],
        out_shape=[
            jax.ShapeDtypeStruct((B, _ROWS, _LANES), jnp.int32),
            jax.ShapeDtypeStruct((B, _NUM_FG, 4), jnp.float32),
        ],
        scratch_shapes=[pltpu.VMEM((_B * _ROWS, _LANES), jnp.float32)
                        for _ in range(5)],
        compiler_params=pltpu.CompilerParams(
            dimension_semantics=("arbitrary",)),
    )(planes, gt_flat)

    labels = labels3.reshape(B, _A_PAD)[:, :_A_REAL].astype(jnp.int8)
    return labels, reg
